# baseline (speedup 1.0000x reference)
import functools

import jax
import jax.numpy as jnp
from jax import lax
from jax.experimental import pallas as pl
from jax.experimental.pallas import tpu as pltpu
from jax.experimental.pallas import tpu_sc as plsc

D = 32
NCHUNK = 4
CHUNK = 128
BPW = NCHUNK * CHUNK


def _gmf_kernel(ux_hbm, ix_hbm, ut_hbm, it_hbm, wb_hbm, out_hbm,
                uidx_v, iidx_v, urows_v, irows_v, wb_v, out_v,
                usem, isem):
    info = plsc.get_sparse_core_info()
    nc = info.num_cores
    wid = lax.axis_index("s") * nc + lax.axis_index("c")

    pltpu.sync_copy(ux_hbm.at[wid], uidx_v)
    pltpu.sync_copy(ix_hbm.at[wid], iidx_v)
    pltpu.sync_copy(wb_hbm, wb_v)

    ucps = [pltpu.async_copy(ut_hbm.at[uidx_v.at[j]],
                             urows_v.at[pl.ds(j * CHUNK, CHUNK)], usem)
            for j in range(NCHUNK)]
    icps = [pltpu.async_copy(it_hbm.at[iidx_v.at[j]],
                             irows_v.at[pl.ds(j * CHUNK, CHUNK)], isem)
            for j in range(NCHUNK)]
    for cp in ucps + icps:
        cp.wait()

    w_lo = wb_v[pl.ds(0, 16)]
    w_hi = wb_v[pl.ds(16, 16)]
    b_vec = wb_v[pl.ds(D, 16)]
    ws = [w_lo[d] for d in range(16)] + [w_hi[d] for d in range(16)]
    bs = b_vec[0]
    lane = lax.iota(jnp.int32, 16)

    def group_body(g, carry):
        rows = g * 16 + lane
        acc = jnp.zeros((16,), jnp.float32)
        for d in range(D):
            col = jnp.full((16,), d, jnp.int32)
            uc = plsc.load_gather(urows_v, [rows, col])
            ic = plsc.load_gather(irows_v, [rows, col])
            acc = acc + uc * ic * ws[d]
        logit = acc + bs
        out_v[pl.ds(g * 16, 16)] = 1.0 / (1.0 + jnp.exp(-logit))
        return carry

    lax.fori_loop(0, BPW // 16, group_body, 0)

    pltpu.sync_copy(out_v, out_hbm.at[wid])


def kernel(user_x, item_x, user_table, item_table, W, b):
    batch = user_x.shape[0]
    info = plsc.get_sparse_core_info()
    nw = info.num_cores * info.num_subcores
    assert batch == nw * BPW

    ux3 = user_x.astype(jnp.int32).reshape(nw, NCHUNK, CHUNK)
    ix3 = item_x.astype(jnp.int32).reshape(nw, NCHUNK, CHUNK)
    wb = jnp.concatenate([W.reshape(D), jnp.broadcast_to(b, (D,))])

    mesh = plsc.VectorSubcoreMesh(core_axis_name="c", subcore_axis_name="s")
    run = functools.partial(
        pl.kernel,
        mesh=mesh,
        compiler_params=pltpu.CompilerParams(
            needs_layout_passes=False, use_tc_tiling_on_sc=False),
        out_type=jax.ShapeDtypeStruct((nw, BPW), jnp.float32),
        scratch_types=[
            pltpu.VMEM((NCHUNK, CHUNK), jnp.int32),
            pltpu.VMEM((NCHUNK, CHUNK), jnp.int32),
            pltpu.VMEM((BPW, D), jnp.float32),
            pltpu.VMEM((BPW, D), jnp.float32),
            pltpu.VMEM((2 * D,), jnp.float32),
            pltpu.VMEM((BPW,), jnp.float32),
            pltpu.SemaphoreType.DMA,
            pltpu.SemaphoreType.DMA,
        ],
    )(_gmf_kernel)
    out = run(ux3, ix3, user_table, item_table, wb)
    return out.reshape(batch, 1)


# speedup vs baseline: 1.0086x; 1.0086x over previous
import functools

import jax
import jax.numpy as jnp
from jax import lax
from jax.experimental import pallas as pl
from jax.experimental.pallas import tpu as pltpu
from jax.experimental.pallas import tpu_sc as plsc

D = 32
PACK = 128 // D
N_PACKED = 250000
BPW = 512
NPASS = 2
PASS_B = BPW // NPASS


def _gmf_kernel(ux_hbm, ix_hbm, ut_hbm, it_hbm, wb_hbm, out_hbm,
                uidx_v, iidx_v, ujrow_v, ijrow_v, urows_v, irows_v,
                wb_v, out_v, usem, isem):
    info = plsc.get_sparse_core_info()
    nc = info.num_cores
    wid = lax.axis_index("s") * nc + lax.axis_index("c")

    pltpu.sync_copy(ux_hbm.at[wid], uidx_v)
    pltpu.sync_copy(ix_hbm.at[wid], iidx_v)
    pltpu.sync_copy(wb_hbm, wb_v)

    def rowid_body(t, carry):
        for h in range(8):
            ujrow_v[t, pl.ds(h * 16, 16)] = (
                uidx_v[t, pl.ds(h * 16, 16)] >> 2)
            ijrow_v[t, pl.ds(h * 16, 16)] = (
                iidx_v[t, pl.ds(h * 16, 16)] >> 2)
        return carry

    lax.fori_loop(0, 4, rowid_body, 0)

    w_lo = wb_v[pl.ds(0, 16)]
    w_hi = wb_v[pl.ds(16, 16)]
    bs = wb_v[pl.ds(D, 16)][0]
    lane = lax.iota(jnp.int32, 16)

    for p in range(NPASS):
        for j in range(PASS_B // 128):
            t = p * (PASS_B // 128) + j
            pltpu.async_copy(ut_hbm.at[ujrow_v.at[t]],
                             urows_v.at[pl.ds(j * 128, 128), :], usem)
            pltpu.async_copy(it_hbm.at[ijrow_v.at[t]],
                             irows_v.at[pl.ds(j * 128, 128), :], isem)
        pltpu.make_async_copy(
            ut_hbm.at[pl.ds(0, PASS_B)], urows_v, usem).wait()
        pltpu.make_async_copy(
            it_hbm.at[pl.ds(0, PASS_B)], irows_v, isem).wait()

        def chunk_body(c, carry, p=p):
            flat = p * PASS_B + c * 16
            t = flat >> 7
            off = flat & 127
            ru = uidx_v[t, pl.ds(off, 16)]
            ri = iidx_v[t, pl.ds(off, 16)]
            res = jnp.zeros((16,), jnp.float32)
            uoff = (ru & 3) * D
            ioff = (ri & 3) * D
            for k in range(16):
                row = c * 16 + k
                uo = uoff[k]
                io = ioff[k]
                u0 = urows_v[row, pl.ds(uo, 16)]
                u1 = urows_v[row, pl.ds(uo + 16, 16)]
                i0 = irows_v[row, pl.ds(io, 16)]
                i1 = irows_v[row, pl.ds(io + 16, 16)]
                s = jnp.sum(u0 * i0 * w_lo + u1 * i1 * w_hi)
                res = jnp.where(lane == k, s, res)
            out_v[pl.ds(p * PASS_B + c * 16, 16)] = (
                1.0 / (1.0 + jnp.exp(-(res + bs))))
            return carry

        lax.fori_loop(0, PASS_B // 16, chunk_body, 0)

    pltpu.sync_copy(out_v, out_hbm.at[wid])


def kernel(user_x, item_x, user_table, item_table, W, b):
    batch = user_x.shape[0]
    info = plsc.get_sparse_core_info()
    nw = info.num_cores * info.num_subcores
    assert batch == nw * BPW

    ux3 = user_x.astype(jnp.int32).reshape(nw, 4, 128)
    ix3 = item_x.astype(jnp.int32).reshape(nw, 4, 128)
    ut_p = user_table.reshape(N_PACKED, PACK * D)
    it_p = item_table.reshape(N_PACKED, PACK * D)
    wb = jnp.concatenate([W.reshape(D), jnp.broadcast_to(b, (D,))])

    mesh = plsc.VectorSubcoreMesh(core_axis_name="c", subcore_axis_name="s")
    run = functools.partial(
        pl.kernel,
        mesh=mesh,
        compiler_params=pltpu.CompilerParams(
            needs_layout_passes=False, use_tc_tiling_on_sc=False),
        out_type=jax.ShapeDtypeStruct((nw, BPW), jnp.float32),
        scratch_types=[
            pltpu.VMEM((4, 128), jnp.int32),
            pltpu.VMEM((4, 128), jnp.int32),
            pltpu.VMEM((4, 128), jnp.int32),
            pltpu.VMEM((4, 128), jnp.int32),
            pltpu.VMEM((PASS_B, PACK * D), jnp.float32),
            pltpu.VMEM((PASS_B, PACK * D), jnp.float32),
            pltpu.VMEM((2 * D,), jnp.float32),
            pltpu.VMEM((BPW,), jnp.float32),
            pltpu.SemaphoreType.DMA,
            pltpu.SemaphoreType.DMA,
        ],
    )(_gmf_kernel)
    out = run(ux3, ix3, ut_p, it_p, wb)
    return out.reshape(batch, 1)
